# Initial kernel scaffold; baseline (speedup 1.0000x reference)
import jax
import jax.numpy as jnp
from jax.experimental import pallas as pl


def kernel(x, edge_index, node_ids, emb_table, W_proj, b_proj, W_self1, W_neigh1, b1, W_self2, W_neigh2, b2, alpha1_logit, alpha2_logit):
    raise NotImplementedError("write your pallas kernel here")


# speedup vs baseline: 4.4732x; 4.4732x over previous
import functools

import jax
import jax.numpy as jnp
from jax import lax
from jax.experimental import pallas as pl
from jax.experimental.pallas import tpu as pltpu
from jax.experimental.pallas import tpu_sc as plsc

N, E, D_IN, H = 10000, 320000, 768, 128

NC, NS, L = 2, 16, 16
NW = NC * NS
CHUNK = 128
CPW0 = 128
CPW1 = 32
NCHUNKS = NS * (CPW0 + CPW1)
E_PAD = NCHUNKS * CHUNK
N_PAD = 10240
ROWS_PER_TILE = N_PAD // NS


SLAB = 8


def _tile_span(c, s):
    base = jnp.where(c == 0, s * CPW0, NS * CPW0 + s * CPW1)
    nslab = jnp.where(c == 0, CPW0 // SLAB, CPW1 // SLAB)
    return base, nslab


def _zero_accumulator(buf_v, agg_sh, r0):
    zero16 = jnp.zeros((L,), jnp.float32)
    lanes = lax.iota(jnp.int32, L)

    def zrow(i, _):
        row = jnp.full((L,), i, jnp.int32)
        for k in range(H // L):
            plsc.store_scatter(buf_v, [row, lanes + k * L], zero16)
        return _
    lax.fori_loop(0, CHUNK, zrow, None)
    for m in range(ROWS_PER_TILE // CHUNK):
        pltpu.sync_copy(buf_v, agg_sh.at[pl.ds(r0 + m * CHUNK, CHUNK)])


def _dump_accumulator(buf_v, idx_v, agg_sh, out_view, r0):
    lanes = lax.iota(jnp.int32, L)
    for m in range(ROWS_PER_TILE // CHUNK):
        rowm = jnp.full((L,), m, jnp.int32)
        for k in range(CHUNK // L):
            plsc.store_scatter(idx_v, [rowm, lanes + k * L],
                               r0 + m * CHUNK + k * L + lanes)
    for m in range(ROWS_PER_TILE // CHUNK):
        pltpu.sync_copy(agg_sh.at[pl.ds(r0 + m * CHUNK, CHUNK)], buf_v)
        pltpu.sync_copy(buf_v, out_view.at[idx_v.at[m]])


def _sc_agg_deg(h_hbm, src_hbm, dst_hbm,
                agg_out, deg_out,
                agg_sh, idxs_v, idxd_v, rows0_v, rows1_v, hist_v, sem):
    c = lax.axis_index("c")
    s = lax.axis_index("s")
    wid = s * NC + c
    r0 = s * ROWS_PER_TILE
    zero16 = jnp.zeros((L,), jnp.float32)
    ones16 = jnp.ones((L,), jnp.float32)

    def zstep(i, _):
        hist_v[pl.ds(i * L, L)] = zero16
        return _
    lax.fori_loop(0, N_PAD // L, zstep, None)

    _zero_accumulator(rows0_v, agg_sh, r0)
    plsc.subcore_barrier()

    bufs = (rows0_v, rows1_v)
    base0, nslab = _tile_span(c, s)

    def slab_step(g, _):
        base = base0 + g * SLAB
        pltpu.sync_copy(src_hbm.at[pl.ds(base, SLAB)], idxs_v)
        pltpu.sync_copy(dst_hbm.at[pl.ds(base, SLAB)], idxd_v)
        pending = pltpu.async_copy(h_hbm.at[idxs_v.at[0]], bufs[0], sem)
        for j in range(SLAB):
            pending.wait()
            if j + 1 < SLAB:
                pending = pltpu.async_copy(h_hbm.at[idxs_v.at[j + 1]],
                                           bufs[(j + 1) % 2], sem)
            pltpu.sync_copy(bufs[j % 2], agg_sh.at[idxd_v.at[j]], add=True)
            for k in range(CHUNK // L):
                idx = idxd_v[j, pl.ds(k * L, L)]
                plsc.addupdate_scatter(hist_v, [idx], ones16)
        return _

    lax.fori_loop(0, nslab, slab_step, None)
    plsc.subcore_barrier()

    _dump_accumulator(rows0_v, idxs_v, agg_sh, agg_out.at[c], r0)
    pltpu.sync_copy(hist_v, deg_out.at[wid])


def _sc_agg_only(h_hbm, src_hbm, dst_hbm,
                 agg_out,
                 agg_sh, idxs_v, idxd_v, rows0_v, rows1_v, sem):
    c = lax.axis_index("c")
    s = lax.axis_index("s")

    r0 = s * ROWS_PER_TILE
    _zero_accumulator(rows0_v, agg_sh, r0)
    plsc.subcore_barrier()

    bufs = (rows0_v, rows1_v)
    base0, nslab = _tile_span(c, s)

    def slab_step(g, _):
        base = base0 + g * SLAB
        pltpu.sync_copy(src_hbm.at[pl.ds(base, SLAB)], idxs_v)
        pltpu.sync_copy(dst_hbm.at[pl.ds(base, SLAB)], idxd_v)
        pending = pltpu.async_copy(h_hbm.at[idxs_v.at[0]], bufs[0], sem)
        for j in range(SLAB):
            pending.wait()
            if j + 1 < SLAB:
                pending = pltpu.async_copy(h_hbm.at[idxs_v.at[j + 1]],
                                           bufs[(j + 1) % 2], sem)
            pltpu.sync_copy(bufs[j % 2], agg_sh.at[idxd_v.at[j]], add=True)
        return _

    lax.fori_loop(0, nslab, slab_step, None)
    plsc.subcore_barrier()

    _dump_accumulator(rows0_v, idxs_v, agg_sh, agg_out.at[c], r0)


@functools.cache
def _sc_mesh():
    return plsc.VectorSubcoreMesh(core_axis_name="c", subcore_axis_name="s",
                                  num_cores=NC, num_subcores=NS)


@jax.jit
def _aggregate_with_deg(h, src2d, dst2d):
    return pl.kernel(
        _sc_agg_deg,
        out_type=(
            jax.ShapeDtypeStruct((NC, N_PAD, H), jnp.float32),
            jax.ShapeDtypeStruct((NW, N_PAD), jnp.float32),
        ),
        mesh=_sc_mesh(),
        compiler_params=pltpu.CompilerParams(needs_layout_passes=False),
        scratch_types=[
            pltpu.VMEM_SHARED((N_PAD, H), jnp.float32),
            pltpu.VMEM((SLAB, CHUNK), jnp.int32),
            pltpu.VMEM((SLAB, CHUNK), jnp.int32),
            pltpu.VMEM((CHUNK, H), jnp.float32),
            pltpu.VMEM((CHUNK, H), jnp.float32),
            pltpu.VMEM((N_PAD,), jnp.float32),
            pltpu.SemaphoreType.DMA,
        ],
    )(h, src2d, dst2d)


@jax.jit
def _aggregate(h, src2d, dst2d):
    return pl.kernel(
        _sc_agg_only,
        out_type=jax.ShapeDtypeStruct((NC, N_PAD, H), jnp.float32),
        mesh=_sc_mesh(),
        compiler_params=pltpu.CompilerParams(needs_layout_passes=False),
        scratch_types=[
            pltpu.VMEM_SHARED((N_PAD, H), jnp.float32),
            pltpu.VMEM((SLAB, CHUNK), jnp.int32),
            pltpu.VMEM((SLAB, CHUNK), jnp.int32),
            pltpu.VMEM((CHUNK, H), jnp.float32),
            pltpu.VMEM((CHUNK, H), jnp.float32),
            pltpu.SemaphoreType.DMA,
        ],
    )(h, src2d, dst2d)


_BN = 400


def _proj_body(x_ref, emb_ref, wp_ref, bp_ref, out_ref):
    out_ref[...] = (emb_ref[...]
                    + jnp.dot(x_ref[...], wp_ref[...],
                              preferred_element_type=jnp.float32)
                    + bp_ref[...])


@jax.jit
def _project(x, emb, wp, bp):
    return pl.pallas_call(
        _proj_body,
        grid=(10,),
        in_specs=[
            pl.BlockSpec((1000, D_IN), lambda i: (i, 0)),
            pl.BlockSpec((1000, H), lambda i: (i, 0)),
            pl.BlockSpec((D_IN, H), lambda i: (0, 0)),
            pl.BlockSpec((1, H), lambda i: (0, 0)),
        ],
        out_specs=pl.BlockSpec((1000, H), lambda i: (i, 0)),
        out_shape=jax.ShapeDtypeStruct((N, H), jnp.float32),
    )(x, emb, wp, bp)


def _combine_body(relu, hp_ref, h0_ref, agg_ref, deg_ref, ws_ref, wn_ref,
                  b_ref, al_ref, out_ref):
    a = jax.nn.sigmoid(al_ref[0, 0])
    aggs = agg_ref[0] + agg_ref[1]
    deg = jnp.sum(deg_ref[...], axis=1, keepdims=True)
    inv = 1.0 / jnp.maximum(deg, 1.0)
    mean = aggs * inv
    out = (jnp.dot(hp_ref[...], ws_ref[...], preferred_element_type=jnp.float32)
           + jnp.dot(mean, wn_ref[...], preferred_element_type=jnp.float32)
           + b_ref[...])
    if relu:
        out = jnp.maximum(out, 0.0)
    out_ref[...] = (1.0 - a) * out + a * h0_ref[...]


@functools.partial(jax.jit, static_argnames=("relu",))
def _combine(hp, h0, agg, deg, ws, wn, b, alogit, relu):
    return pl.pallas_call(
        functools.partial(_combine_body, relu),
        grid=(N // _BN,),
        in_specs=[
            pl.BlockSpec((_BN, H), lambda i: (i, 0)),
            pl.BlockSpec((_BN, H), lambda i: (i, 0)),
            pl.BlockSpec((NC, _BN, H), lambda i: (0, i, 0)),
            pl.BlockSpec((_BN, NW), lambda i: (i, 0)),
            pl.BlockSpec((H, H), lambda i: (0, 0)),
            pl.BlockSpec((H, H), lambda i: (0, 0)),
            pl.BlockSpec((1, H), lambda i: (0, 0)),
            pl.BlockSpec((1, 1), lambda i: (0, 0)),
        ],
        out_specs=pl.BlockSpec((_BN, H), lambda i: (i, 0)),
        out_shape=jax.ShapeDtypeStruct((N, H), jnp.float32),
    )(hp, h0, agg, deg, ws, wn, b, alogit)


def kernel(x, edge_index, node_ids, emb_table, W_proj, b_proj,
           W_self1, W_neigh1, b1, W_self2, W_neigh2, b2,
           alpha1_logit, alpha2_logit):
    src = edge_index[0].astype(jnp.int32)
    dst = edge_index[1].astype(jnp.int32)
    pad = E_PAD - E
    src2d = jnp.concatenate(
        [src, jnp.zeros((pad,), jnp.int32)]).reshape(NCHUNKS, CHUNK)
    dst2d = jnp.concatenate(
        [dst, jnp.full((pad,), N, jnp.int32)]).reshape(NCHUNKS, CHUNK)

    bp = b_proj.reshape(1, H)
    b1r = b1.reshape(1, H)
    b2r = b2.reshape(1, H)
    a1 = jnp.asarray(alpha1_logit, jnp.float32).reshape(1, 1)
    a2 = jnp.asarray(alpha2_logit, jnp.float32).reshape(1, 1)

    h0 = _project(x, emb_table, W_proj, bp)
    agg1, deg = _aggregate_with_deg(h0, src2d, dst2d)
    degT = deg.T
    h1 = _combine(h0, h0, agg1, degT, W_self1, W_neigh1, b1r, a1, relu=True)
    agg2 = _aggregate(h1, src2d, dst2d)
    h2 = _combine(h1, h0, agg2, degT, W_self2, W_neigh2, b2r, a2, relu=False)
    return h2
